# baseline (device time: 49022 ns/iter reference)
import os

import jax
import jax.numpy as jnp
from jax import lax
from jax.experimental import pallas as pl
from jax.experimental.pallas import tpu as pltpu

_BENCH = os.environ.get("BENCH", "")

N_DEV = 4
B, SQ, D = 4, 256, 1024
DH = 128
SCALE = 0.08838834764831843
ROWS = B * SQ


def kernel(x, Wq, Wo, Wk, Wv):
    n_heads = Wq.shape[1] // DH
    x2 = x.reshape(ROWS, D)

    def body(x_ref, wq_ref, wo_ref, wk_ref, wv_ref, out_ref,
             sbuf_ref, rbuf_ref, send_sems, recv_sems):
        my = lax.axis_index("i")

        barrier_sem = pltpu.get_barrier_semaphore()
        for k in range(1, N_DEV):
            pl.semaphore_signal(
                barrier_sem, inc=1,
                device_id=(lax.rem(my + k, N_DEV),),
                device_id_type=pl.DeviceIdType.MESH,
            )
        pl.semaphore_wait(barrier_sem, N_DEV - 1)

        wq16 = wq_ref[...].astype(jnp.bfloat16)
        wk16 = wk_ref[...].astype(jnp.bfloat16)
        wv16 = wv_ref[...].astype(jnp.bfloat16)
        wo16 = wo_ref[...].astype(jnp.bfloat16)

        def compute_batch(bi, send_slot=None):
            r = pl.ds(bi * SQ, SQ)
            xb = x_ref[r, :].astype(jnp.bfloat16)
            qm = jnp.dot(xb, wq16, preferred_element_type=jnp.float32)
            km = jnp.dot(xb, wk16, preferred_element_type=jnp.float32)
            vm = jnp.dot(xb, wv16, preferred_element_type=jnp.float32)
            outs = []
            for h in range(n_heads):
                c0 = h * DH
                qh = qm[:, c0:c0 + DH]
                kh = km[:, c0:c0 + DH]
                vh = vm[:, c0:c0 + DH]
                s = lax.dot_general(
                    qh, kh, (((1,), (1,)), ((), ())),
                    preferred_element_type=jnp.float32,
                ) * SCALE
                m = jnp.max(s, axis=-1, keepdims=True)
                p = jnp.exp(s - m)
                l = jnp.sum(p, axis=-1, keepdims=True)
                outs.append(
                    jnp.dot(p, vh, preferred_element_type=jnp.float32) / l
                )
            attn_b = jnp.concatenate(outs, axis=1).astype(jnp.bfloat16)
            out_b = jnp.dot(
                attn_b, wo16, preferred_element_type=jnp.float32
            )
            out_ref[r, :] = out_b
            if send_slot is not None:
                sbuf_ref[send_slot, :, :] = out_b.astype(jnp.bfloat16)

        if _BENCH == "compute":
            for bi in range(B):
                compute_batch(bi)
            return

        def send(src_slot, dst_slot, k, sem):
            rdma = pltpu.make_async_remote_copy(
                src_ref=sbuf_ref.at[src_slot],
                dst_ref=rbuf_ref.at[dst_slot],
                send_sem=send_sems.at[sem],
                recv_sem=recv_sems.at[sem],
                device_id=(lax.rem(my + k, N_DEV),),
                device_id_type=pl.DeviceIdType.MESH,
            )
            rdma.start()
            return rdma

        rs = []
        for k in range(1, N_DEV):
            compute_batch(lax.rem(my + k, N_DEV), send_slot=k - 1)
            rs.append(send(k - 1, k - 1, k, k - 1))
        compute_batch(my)

        for r in rs:
            r.wait()
        rows = pl.ds(my * SQ, SQ)
        out_ref[rows, :] = (
            out_ref[rows, :]
            + rbuf_ref[0].astype(jnp.float32)
            + rbuf_ref[1].astype(jnp.float32)
            + rbuf_ref[2].astype(jnp.float32)
        )
        sbuf_ref[3, :, :] = out_ref[rows, :].astype(jnp.bfloat16)

        ag = [send(3, 6 - k, k, 2 + k) for k in range(1, N_DEV)]

        ag[2].wait()
        out_ref[pl.ds(lax.rem(my + 1, N_DEV) * SQ, SQ), :] = (
            rbuf_ref[3].astype(jnp.float32)
        )
        ag[1].wait()
        out_ref[pl.ds(lax.rem(my + 2, N_DEV) * SQ, SQ), :] = (
            rbuf_ref[4].astype(jnp.float32)
        )
        ag[0].wait()
        out_ref[pl.ds(lax.rem(my + 3, N_DEV) * SQ, SQ), :] = (
            rbuf_ref[5].astype(jnp.float32)
        )

    out2 = pl.pallas_call(
        body,
        out_shape=jax.ShapeDtypeStruct((ROWS, D), jnp.float32),
        in_specs=[pl.BlockSpec(memory_space=pltpu.VMEM)] * 5,
        out_specs=pl.BlockSpec(memory_space=pltpu.VMEM),
        scratch_shapes=[
            pltpu.VMEM((4, SQ, D), jnp.bfloat16),
            pltpu.VMEM((6, SQ, D), jnp.bfloat16),
            pltpu.SemaphoreType.DMA((6,)),
            pltpu.SemaphoreType.DMA((6,)),
        ],
        compiler_params=pltpu.CompilerParams(collective_id=0),
    )(x2, Wq, Wo, Wk, Wv)
    return out2.reshape(B, SQ, D)


# device time: 45491 ns/iter; 1.0776x vs baseline; 1.0776x over previous
import os

import jax
import jax.numpy as jnp
from jax import lax
from jax.experimental import pallas as pl
from jax.experimental.pallas import tpu as pltpu

_BENCH = os.environ.get("BENCH", "")

N_DEV = 4
B, SQ, D = 4, 256, 1024
DH = 128
SCALE = 0.08838834764831843
ROWS = B * SQ
U = 128


def kernel(x, Wq, Wo, Wk, Wv):
    n_heads = Wq.shape[1] // DH
    x2 = x.reshape(ROWS, D)

    def body(x_ref, wq_ref, wo_ref, wk_ref, wv_ref, out_ref,
             sbuf_ref, rbuf_ref, send_sems, recv_sems):
        my = lax.axis_index("i")
        a_half = (my ^ (my >> 1)) & 1
        b_half = (my >> 1) & 1
        qa_sub = b_half
        qb_sub = my & 1
        p1 = my ^ 1
        p2 = 3 - my

        barrier_sem = pltpu.get_barrier_semaphore()
        for nbr in (p1, p2):
            pl.semaphore_signal(
                barrier_sem, inc=1,
                device_id=(nbr,), device_id_type=pl.DeviceIdType.MESH,
            )
        pl.semaphore_wait(barrier_sem, 2)

        wq16 = wq_ref[...].astype(jnp.bfloat16)
        wk16 = wk_ref[...].astype(jnp.bfloat16)
        wv16 = wv_ref[...].astype(jnp.bfloat16)
        wo16 = wo_ref[...].astype(jnp.bfloat16)

        def compute_batch(bi, send_slot=None):
            r = pl.ds(bi * SQ, SQ)
            xb = x_ref[r, :].astype(jnp.bfloat16)
            qm = jnp.dot(xb, wq16, preferred_element_type=jnp.float32)
            km = jnp.dot(xb, wk16, preferred_element_type=jnp.float32)
            vm = jnp.dot(xb, wv16, preferred_element_type=jnp.float32)
            outs = []
            for h in range(n_heads):
                c0 = h * DH
                qh = qm[:, c0:c0 + DH]
                kh = km[:, c0:c0 + DH]
                vh = vm[:, c0:c0 + DH]
                s = lax.dot_general(
                    qh, kh, (((1,), (1,)), ((), ())),
                    preferred_element_type=jnp.float32,
                ) * SCALE
                m = jnp.max(s, axis=-1, keepdims=True)
                p = jnp.exp(s - m)
                l = jnp.sum(p, axis=-1, keepdims=True)
                outs.append(
                    jnp.dot(p, vh, preferred_element_type=jnp.float32) / l
                )
            attn_b = jnp.concatenate(outs, axis=1).astype(jnp.bfloat16)
            out_b = jnp.dot(
                attn_b, wo16, preferred_element_type=jnp.float32
            )
            out_ref[r, :] = out_b
            if send_slot is not None:
                sbuf_ref[send_slot, :, :] = out_b.astype(jnp.bfloat16)

        def exch(slot, n_u, partner, sem):
            rdma = pltpu.make_async_remote_copy(
                src_ref=sbuf_ref.at[slot, pl.ds(0, n_u * U), :],
                dst_ref=rbuf_ref.at[slot, pl.ds(0, n_u * U), :],
                send_sem=send_sems.at[sem],
                recv_sem=recv_sems.at[sem],
                device_id=(partner,),
                device_id_type=pl.DeviceIdType.MESH,
            )
            rdma.start()
            return rdma

        def stage(slot, src_u, n_u):
            rows = pl.ds(src_u * U, n_u * U)
            sbuf_ref[slot, pl.ds(0, n_u * U), :] = (
                out_ref[rows, :].astype(jnp.bfloat16)
            )

        def accum(slot, dst_u, n_u):
            rows = pl.ds(dst_u * U, n_u * U)
            out_ref[rows, :] = out_ref[rows, :] + (
                rbuf_ref[slot, pl.ds(0, n_u * U), :].astype(jnp.float32)
            )

        def store(slot, dst_u, n_u):
            rows = pl.ds(dst_u * U, n_u * U)
            out_ref[rows, :] = (
                rbuf_ref[slot, pl.ds(0, n_u * U), :].astype(jnp.float32)
            )

        if _BENCH == "compute":
            for bi in range(B):
                compute_batch(bi)
            return
        if _BENCH == "matmul":
            f8 = jnp.float8_e4m3fn
            wq8 = (wq_ref[...] * 50.0).astype(f8)
            wk8 = (wk_ref[...] * 50.0).astype(f8)
            wv8 = (wv_ref[...] * 50.0).astype(f8)
            wo8 = (wo_ref[...] * 50.0).astype(f8)
            for bi in range(B):
                r = pl.ds(bi * SQ, SQ)
                xb = x_ref[r, :].astype(f8)
                qm = jnp.dot(xb, wq8, preferred_element_type=jnp.float32)
                km = jnp.dot(xb, wk8, preferred_element_type=jnp.float32)
                vm = jnp.dot(xb, wv8, preferred_element_type=jnp.float32)
                acc = ((qm + km + vm) * 0.02).astype(f8)
                out_ref[r, :] = jnp.dot(
                    acc, wo8, preferred_element_type=jnp.float32
                )
            return
        if _BENCH == "comm":
            out_ref[...] = jnp.zeros((ROWS, D), jnp.float32)
            stage(0, 2 * (1 - a_half), 2)
            ra = exch(0, 2, p1, 0)
            stage(1, 4 + 2 * (1 - b_half), 2)
            rb = exch(1, 2, p2, 1)
        def accum_stage(rslot, dst_u, sslot):
            rows = pl.ds(dst_u * U, 2 * U)
            val = out_ref[rows, :] + (
                rbuf_ref[rslot, :, :].astype(jnp.float32)
            )
            out_ref[rows, :] = val
            sbuf_ref[sslot, :, :] = val.astype(jnp.bfloat16)

        ua = 2 * a_half
        ub = 4 + 2 * b_half
        if _BENCH == "comm":
            ra.wait()
            accum_stage(0, ua, 2)
            ra = exch(2, 2, p2, 2)
        else:
            compute_batch(1 - a_half, send_slot=0)
            ra = exch(0, 2, p1, 0)
            compute_batch(3 - b_half, send_slot=1)
            rb = exch(1, 2, p2, 1)
            compute_batch(a_half)
            ra.wait()
            accum_stage(0, ua, 2)
            ra = exch(2, 2, p2, 2)
            compute_batch(2 + b_half)

        rb.wait()
        accum_stage(1, ub, 3)
        rb = exch(3, 2, p1, 3)

        ra.wait()
        accum_stage(2, ua, 4)
        ra = exch(4, 2, p1, 4)
        rb.wait()
        accum_stage(3, ub, 5)
        rb = exch(5, 2, p2, 5)

        ra.wait()
        store(4, 2 * (1 - a_half), 2)
        rb.wait()
        store(5, 4 + 2 * (1 - b_half), 2)

    out2 = pl.pallas_call(
        body,
        out_shape=jax.ShapeDtypeStruct((ROWS, D), jnp.float32),
        in_specs=[pl.BlockSpec(memory_space=pltpu.VMEM)] * 5,
        out_specs=pl.BlockSpec(memory_space=pltpu.VMEM),
        scratch_shapes=[
            pltpu.VMEM((8, 2 * U, D), jnp.bfloat16),
            pltpu.VMEM((8, 2 * U, D), jnp.bfloat16),
            pltpu.SemaphoreType.DMA((8,)),
            pltpu.SemaphoreType.DMA((8,)),
        ],
        compiler_params=pltpu.CompilerParams(collective_id=0),
    )(x2, Wq, Wo, Wk, Wv)
    return out2.reshape(B, SQ, D)


# device time: 45233 ns/iter; 1.0838x vs baseline; 1.0057x over previous
import os

import jax
import jax.numpy as jnp
from jax import lax
from jax.experimental import pallas as pl
from jax.experimental.pallas import tpu as pltpu

_BENCH = os.environ.get("BENCH", "")

N_DEV = 4
B, SQ, D = 4, 256, 1024
DH = 128
SCALE = 0.08838834764831843
ROWS = B * SQ
U = 128


def kernel(x, Wq, Wo, Wk, Wv):
    n_heads = Wq.shape[1] // DH
    x2 = x.reshape(ROWS, D)

    def body(x_ref, wq_ref, wo_ref, wk_ref, wv_ref, out_ref,
             sbuf_ref, rbuf_ref, send_sems, recv_sems):
        my = lax.axis_index("i")
        a_half = (my ^ (my >> 1)) & 1
        b_half = (my >> 1) & 1
        qa_sub = b_half
        qb_sub = my & 1
        p1 = my ^ 1
        p2 = 3 - my

        barrier_sem = pltpu.get_barrier_semaphore()
        for nbr in (p1, p2):
            pl.semaphore_signal(
                barrier_sem, inc=1,
                device_id=(nbr,), device_id_type=pl.DeviceIdType.MESH,
            )
        pl.semaphore_wait(barrier_sem, 2)

        wq16 = wq_ref[...].astype(jnp.bfloat16)
        wk16 = wk_ref[...].astype(jnp.bfloat16)
        wv16 = wv_ref[...].astype(jnp.bfloat16)
        wo16 = wo_ref[...].astype(jnp.bfloat16)

        def compute_batch(bi, send_slot=None):
            r = pl.ds(bi * SQ, SQ)
            xb = x_ref[r, :].astype(jnp.bfloat16)
            qm = jnp.dot(xb, wq16, preferred_element_type=jnp.float32)
            km = jnp.dot(xb, wk16, preferred_element_type=jnp.float32)
            vm = jnp.dot(xb, wv16, preferred_element_type=jnp.float32)
            outs = []
            for h in range(n_heads):
                c0 = h * DH
                qh = qm[:, c0:c0 + DH]
                kh = km[:, c0:c0 + DH]
                vh = vm[:, c0:c0 + DH]
                s = lax.dot_general(
                    qh, kh, (((1,), (1,)), ((), ())),
                    preferred_element_type=jnp.float32,
                ) * SCALE
                m = jnp.max(s, axis=-1, keepdims=True)
                p = jnp.exp(s - m)
                l = jnp.sum(p, axis=-1, keepdims=True)
                outs.append(
                    jnp.dot(p, vh, preferred_element_type=jnp.float32) / l
                )
            attn_b = jnp.concatenate(outs, axis=1).astype(jnp.bfloat16)
            out_b = jnp.dot(
                attn_b, wo16, preferred_element_type=jnp.float32
            )
            out_ref[r, :] = out_b
            if send_slot is not None:
                sbuf_ref[send_slot, :, :] = out_b.astype(jnp.bfloat16)

        def compute_half(bi, half, kv=None):
            if kv is None:
                xbf = x_ref[pl.ds(bi * SQ, SQ), :].astype(jnp.bfloat16)
                km = jnp.dot(xbf, wk16, preferred_element_type=jnp.float32)
                vm = jnp.dot(xbf, wv16, preferred_element_type=jnp.float32)
            else:
                km, vm = kv
            r = pl.ds(bi * SQ + half * U, U)
            xh = x_ref[r, :].astype(jnp.bfloat16)
            qm = jnp.dot(xh, wq16, preferred_element_type=jnp.float32)
            outs = []
            for h in range(n_heads):
                c0 = h * DH
                qh = qm[:, c0:c0 + DH]
                kh = km[:, c0:c0 + DH]
                vh = vm[:, c0:c0 + DH]
                s = lax.dot_general(
                    qh, kh, (((1,), (1,)), ((), ())),
                    preferred_element_type=jnp.float32,
                ) * SCALE
                m = jnp.max(s, axis=-1, keepdims=True)
                p = jnp.exp(s - m)
                l = jnp.sum(p, axis=-1, keepdims=True)
                outs.append(
                    jnp.dot(p, vh, preferred_element_type=jnp.float32) / l
                )
            attn_h = jnp.concatenate(outs, axis=1).astype(jnp.bfloat16)
            out_ref[r, :] = jnp.dot(
                attn_h, wo16, preferred_element_type=jnp.float32
            )
            return (km, vm)

        def exch(slot, n_u, partner, sem):
            rdma = pltpu.make_async_remote_copy(
                src_ref=sbuf_ref.at[slot, pl.ds(0, n_u * U), :],
                dst_ref=rbuf_ref.at[slot, pl.ds(0, n_u * U), :],
                send_sem=send_sems.at[sem],
                recv_sem=recv_sems.at[sem],
                device_id=(partner,),
                device_id_type=pl.DeviceIdType.MESH,
            )
            rdma.start()
            return rdma

        def stage(slot, src_u, n_u):
            rows = pl.ds(src_u * U, n_u * U)
            sbuf_ref[slot, pl.ds(0, n_u * U), :] = (
                out_ref[rows, :].astype(jnp.bfloat16)
            )

        def accum(slot, dst_u, n_u):
            rows = pl.ds(dst_u * U, n_u * U)
            out_ref[rows, :] = out_ref[rows, :] + (
                rbuf_ref[slot, pl.ds(0, n_u * U), :].astype(jnp.float32)
            )

        def store(slot, dst_u, n_u):
            rows = pl.ds(dst_u * U, n_u * U)
            out_ref[rows, :] = (
                rbuf_ref[slot, pl.ds(0, n_u * U), :].astype(jnp.float32)
            )

        if _BENCH == "compute":
            for bi in range(B):
                compute_batch(bi)
            return
        if _BENCH == "matmul":
            f8 = jnp.float8_e4m3fn
            wq8 = (wq_ref[...] * 50.0).astype(f8)
            wk8 = (wk_ref[...] * 50.0).astype(f8)
            wv8 = (wv_ref[...] * 50.0).astype(f8)
            wo8 = (wo_ref[...] * 50.0).astype(f8)
            for bi in range(B):
                r = pl.ds(bi * SQ, SQ)
                xb = x_ref[r, :].astype(f8)
                qm = jnp.dot(xb, wq8, preferred_element_type=jnp.float32)
                km = jnp.dot(xb, wk8, preferred_element_type=jnp.float32)
                vm = jnp.dot(xb, wv8, preferred_element_type=jnp.float32)
                acc = ((qm + km + vm) * 0.02).astype(f8)
                out_ref[r, :] = jnp.dot(
                    acc, wo8, preferred_element_type=jnp.float32
                )
            return
        def accum_stage(rslot, dst_u, sslot):
            rows = pl.ds(dst_u * U, 2 * U)
            val = out_ref[rows, :] + (
                rbuf_ref[rslot, :, :].astype(jnp.float32)
            )
            out_ref[rows, :] = val
            sbuf_ref[sslot, :, :] = val.astype(jnp.bfloat16)

        def accum_stage_u(rslot, rsub, dst_u, sslot):
            rows = pl.ds(dst_u * U, U)
            val = out_ref[rows, :] + (
                rbuf_ref[rslot, pl.ds(rsub * U, U), :].astype(jnp.float32)
            )
            out_ref[rows, :] = val
            sbuf_ref[sslot, pl.ds(rsub * U, U), :] = val.astype(jnp.bfloat16)

        def exch_u(slot, sub, partner, sem):
            rdma = pltpu.make_async_remote_copy(
                src_ref=sbuf_ref.at[slot, pl.ds(sub * U, U), :],
                dst_ref=rbuf_ref.at[slot, pl.ds(sub * U, U), :],
                send_sem=send_sems.at[sem],
                recv_sem=recv_sems.at[sem],
                device_id=(partner,),
                device_id_type=pl.DeviceIdType.MESH,
            )
            rdma.start()
            return rdma

        ua = 2 * a_half
        ub = 4 + 2 * b_half

        compute_batch(1 - a_half, send_slot=0)
        ra = exch(0, 2, p1, 0)
        compute_batch(3 - b_half, send_slot=1)
        rb = exch(1, 2, p2, 1)
        compute_batch(a_half)
        ra.wait()
        accum_stage(0, ua, 2)
        ra = exch(2, 2, p2, 2)

        kv = compute_half(2 + b_half, 0)
        rb.wait()
        accum_stage_u(1, 0, ub, 3)
        rb2a = exch_u(3, 0, p1, 3)
        compute_half(2 + b_half, 1, kv)
        accum_stage_u(1, 1, ub + 1, 3)
        rb2b = exch_u(3, 1, p1, 6)

        ra.wait()
        accum_stage(2, ua, 4)
        ra = exch(4, 2, p1, 4)
        rb2a.wait()
        rb2b.wait()
        accum_stage(3, ub, 5)
        rb = exch(5, 2, p2, 5)

        ra.wait()
        store(4, 2 * (1 - a_half), 2)
        rb.wait()
        store(5, 4 + 2 * (1 - b_half), 2)

    out2 = pl.pallas_call(
        body,
        out_shape=jax.ShapeDtypeStruct((ROWS, D), jnp.float32),
        in_specs=[pl.BlockSpec(memory_space=pltpu.VMEM)] * 5,
        out_specs=pl.BlockSpec(memory_space=pltpu.VMEM),
        scratch_shapes=[
            pltpu.VMEM((8, 2 * U, D), jnp.bfloat16),
            pltpu.VMEM((8, 2 * U, D), jnp.bfloat16),
            pltpu.SemaphoreType.DMA((8,)),
            pltpu.SemaphoreType.DMA((8,)),
        ],
        compiler_params=pltpu.CompilerParams(collective_id=0),
    )(x2, Wq, Wo, Wk, Wv)
    return out2.reshape(B, SQ, D)


# device time: 43323 ns/iter; 1.1315x vs baseline; 1.0441x over previous
import os

import jax
import jax.numpy as jnp
from jax import lax
from jax.experimental import pallas as pl
from jax.experimental.pallas import tpu as pltpu

_BENCH = os.environ.get("BENCH", "")

N_DEV = 4
B, SQ, D = 4, 256, 1024
DH = 128
SCALE = 0.08838834764831843
ROWS = B * SQ
U = 128


def kernel(x, Wq, Wo, Wk, Wv):
    n_heads = Wq.shape[1] // DH
    x2 = x.reshape(ROWS, D)

    def body(x_ref, wq_ref, wo_ref, wk_ref, wv_ref, out_ref,
             sbuf_ref, rbuf_ref, send_sems, recv_sems):
        my = lax.axis_index("i")
        a_half = (my ^ (my >> 1)) & 1
        b_half = (my >> 1) & 1
        qa_sub = b_half
        qb_sub = my & 1
        p1 = my ^ 1
        p2 = 3 - my

        barrier_sem = pltpu.get_barrier_semaphore()
        for nbr in (p1, p2):
            pl.semaphore_signal(
                barrier_sem, inc=1,
                device_id=(nbr,), device_id_type=pl.DeviceIdType.MESH,
            )
        pl.semaphore_wait(barrier_sem, 2)

        wq16 = wq_ref[...].astype(jnp.bfloat16)
        wk16 = wk_ref[...].astype(jnp.bfloat16)
        wv16 = wv_ref[...].astype(jnp.bfloat16)
        wo16 = wo_ref[...].astype(jnp.bfloat16)

        def compute_batch(bi, send_slot=None):
            r = pl.ds(bi * SQ, SQ)
            xb = x_ref[r, :].astype(jnp.bfloat16)
            qm = jnp.dot(xb, wq16, preferred_element_type=jnp.float32)
            km = jnp.dot(xb, wk16, preferred_element_type=jnp.float32)
            vm = jnp.dot(xb, wv16, preferred_element_type=jnp.float32)
            outs = []
            for h in range(n_heads):
                c0 = h * DH
                qh = qm[:, c0:c0 + DH]
                kh = km[:, c0:c0 + DH]
                vh = vm[:, c0:c0 + DH]
                s = lax.dot_general(
                    qh, kh, (((1,), (1,)), ((), ())),
                    preferred_element_type=jnp.float32,
                ) * SCALE
                m = jnp.max(s, axis=-1, keepdims=True)
                p = jnp.exp(s - m)
                l = jnp.sum(p, axis=-1, keepdims=True)
                outs.append(
                    jnp.dot(p, vh, preferred_element_type=jnp.float32) / l
                )
            attn_b = jnp.concatenate(outs, axis=1).astype(jnp.bfloat16)
            out_b = jnp.dot(
                attn_b, wo16, preferred_element_type=jnp.float32
            )
            out_ref[r, :] = out_b
            if send_slot is not None:
                sbuf_ref[send_slot, :, :] = out_b.astype(jnp.bfloat16)

        def compute_half(bi, half, kv=None):
            if kv is None:
                xbf = x_ref[pl.ds(bi * SQ, SQ), :].astype(jnp.bfloat16)
                km = jnp.dot(xbf, wk16, preferred_element_type=jnp.float32)
                vm = jnp.dot(xbf, wv16, preferred_element_type=jnp.float32)
            else:
                km, vm = kv
            r = pl.ds(bi * SQ + half * U, U)
            xh = x_ref[r, :].astype(jnp.bfloat16)
            qm = jnp.dot(xh, wq16, preferred_element_type=jnp.float32)
            outs = []
            for h in range(n_heads):
                c0 = h * DH
                qh = qm[:, c0:c0 + DH]
                kh = km[:, c0:c0 + DH]
                vh = vm[:, c0:c0 + DH]
                s = lax.dot_general(
                    qh, kh, (((1,), (1,)), ((), ())),
                    preferred_element_type=jnp.float32,
                ) * SCALE
                m = jnp.max(s, axis=-1, keepdims=True)
                p = jnp.exp(s - m)
                l = jnp.sum(p, axis=-1, keepdims=True)
                outs.append(
                    jnp.dot(p, vh, preferred_element_type=jnp.float32) / l
                )
            attn_h = jnp.concatenate(outs, axis=1).astype(jnp.bfloat16)
            out_ref[r, :] = jnp.dot(
                attn_h, wo16, preferred_element_type=jnp.float32
            )
            return (km, vm)

        def exch(slot, n_u, partner, sem):
            rdma = pltpu.make_async_remote_copy(
                src_ref=sbuf_ref.at[slot, pl.ds(0, n_u * U), :],
                dst_ref=rbuf_ref.at[slot, pl.ds(0, n_u * U), :],
                send_sem=send_sems.at[sem],
                recv_sem=recv_sems.at[sem],
                device_id=(partner,),
                device_id_type=pl.DeviceIdType.MESH,
            )
            rdma.start()
            return rdma

        def stage(slot, src_u, n_u):
            rows = pl.ds(src_u * U, n_u * U)
            sbuf_ref[slot, pl.ds(0, n_u * U), :] = (
                out_ref[rows, :].astype(jnp.bfloat16)
            )

        def accum(slot, dst_u, n_u):
            rows = pl.ds(dst_u * U, n_u * U)
            out_ref[rows, :] = out_ref[rows, :] + (
                rbuf_ref[slot, pl.ds(0, n_u * U), :].astype(jnp.float32)
            )

        def store(slot, dst_u, n_u):
            rows = pl.ds(dst_u * U, n_u * U)
            out_ref[rows, :] = (
                rbuf_ref[slot, pl.ds(0, n_u * U), :].astype(jnp.float32)
            )

        if _BENCH == "compute":
            for bi in range(B):
                compute_batch(bi)
            return
        if _BENCH == "matmul":
            f8 = jnp.float8_e4m3fn
            wq8 = (wq_ref[...] * 50.0).astype(f8)
            wk8 = (wk_ref[...] * 50.0).astype(f8)
            wv8 = (wv_ref[...] * 50.0).astype(f8)
            wo8 = (wo_ref[...] * 50.0).astype(f8)
            for bi in range(B):
                r = pl.ds(bi * SQ, SQ)
                xb = x_ref[r, :].astype(f8)
                qm = jnp.dot(xb, wq8, preferred_element_type=jnp.float32)
                km = jnp.dot(xb, wk8, preferred_element_type=jnp.float32)
                vm = jnp.dot(xb, wv8, preferred_element_type=jnp.float32)
                acc = ((qm + km + vm) * 0.02).astype(f8)
                out_ref[r, :] = jnp.dot(
                    acc, wo8, preferred_element_type=jnp.float32
                )
            return
        def accum_stage(rslot, dst_u, sslot):
            rows = pl.ds(dst_u * U, 2 * U)
            val = out_ref[rows, :] + (
                rbuf_ref[rslot, :, :].astype(jnp.float32)
            )
            out_ref[rows, :] = val
            sbuf_ref[sslot, :, :] = val.astype(jnp.bfloat16)

        def accum_stage_u(rslot, rsub, dst_u, sslot):
            rows = pl.ds(dst_u * U, U)
            val = out_ref[rows, :] + (
                rbuf_ref[rslot, pl.ds(rsub * U, U), :].astype(jnp.float32)
            )
            out_ref[rows, :] = val
            sbuf_ref[sslot, pl.ds(rsub * U, U), :] = val.astype(jnp.bfloat16)

        def exch_u(slot, sub, partner, sem):
            rdma = pltpu.make_async_remote_copy(
                src_ref=sbuf_ref.at[slot, pl.ds(sub * U, U), :],
                dst_ref=rbuf_ref.at[slot, pl.ds(sub * U, U), :],
                send_sem=send_sems.at[sem],
                recv_sem=recv_sems.at[sem],
                device_id=(partner,),
                device_id_type=pl.DeviceIdType.MESH,
            )
            rdma.start()
            return rdma

        ua = 2 * a_half
        ub = 4 + 2 * b_half

        compute_batch(1 - a_half, send_slot=0)
        ra = exch(0, 2, p1, 0)
        compute_batch(3 - b_half, send_slot=1)
        rb = exch(1, 2, p2, 1)
        compute_batch(a_half)
        ra.wait()
        accum_stage(0, ua, 2)
        ra = exch(2, 2, p2, 2)

        kv = compute_half(2 + b_half, 0)
        rb.wait()
        accum_stage_u(1, 0, ub, 3)
        rb2a = exch_u(3, 0, p1, 3)
        compute_half(2 + b_half, 1, kv)
        accum_stage_u(1, 1, ub + 1, 3)
        rb2b = exch_u(3, 1, p1, 6)

        ra.wait()
        accum_stage(2, ua, 4)
        ra = exch(4, 2, p1, 4)
        rb2a.wait()
        accum_stage_u(3, 0, ub, 5)
        rb3a = exch_u(5, 0, p2, 5)
        rb2b.wait()
        accum_stage_u(3, 1, ub + 1, 5)
        rb3b = exch_u(5, 1, p2, 7)

        ra.wait()
        store(4, 2 * (1 - a_half), 2)
        rb3a.wait()
        rb3b.wait()
        store(5, 4 + 2 * (1 - b_half), 2)

    out2 = pl.pallas_call(
        body,
        out_shape=jax.ShapeDtypeStruct((ROWS, D), jnp.float32),
        in_specs=[pl.BlockSpec(memory_space=pltpu.VMEM)] * 5,
        out_specs=pl.BlockSpec(memory_space=pltpu.VMEM),
        scratch_shapes=[
            pltpu.VMEM((8, 2 * U, D), jnp.bfloat16),
            pltpu.VMEM((8, 2 * U, D), jnp.bfloat16),
            pltpu.SemaphoreType.DMA((8,)),
            pltpu.SemaphoreType.DMA((8,)),
        ],
        compiler_params=pltpu.CompilerParams(collective_id=0),
    )(x2, Wq, Wo, Wk, Wv)
    return out2.reshape(B, SQ, D)
